# baseline (speedup 1.0000x reference)
import jax
import jax.numpy as jnp
from jax import lax
from jax.experimental import pallas as pl
from jax.experimental.pallas import tpu as pltpu

ENCODING_DIM = 64
LANES = 128


def _body(s_ref, blk, out_v):
    r = s_ref[0] % LANES
    lane = lax.broadcasted_iota(jnp.int32, (ENCODING_DIM, LANES), 1)
    sel = jnp.where(lane == r, blk[...], 0.0)
    out_v[...] = jnp.sum(sel, axis=1)


def kernel(position, num_players, table):
    s = jnp.asarray(position, jnp.int32) % jnp.asarray(num_players, jnp.int32)
    s_arr = jnp.reshape(s, (1,))
    table_t = table.T
    grid_spec = pltpu.PrefetchScalarGridSpec(
        num_scalar_prefetch=1,
        grid=(1,),
        in_specs=[
            pl.BlockSpec(
                (ENCODING_DIM, LANES), lambda i, s_ref: (0, s_ref[0] // LANES)
            )
        ],
        out_specs=pl.BlockSpec((ENCODING_DIM,), lambda i, s_ref: (0,)),
    )
    out = pl.pallas_call(
        _body,
        grid_spec=grid_spec,
        out_shape=jax.ShapeDtypeStruct((ENCODING_DIM,), jnp.float32),
    )(s_arr, table_t)
    return out


# speedup vs baseline: 1.0026x; 1.0026x over previous
import jax
import jax.numpy as jnp
from jax import lax
from jax.experimental import pallas as pl
from jax.experimental.pallas import tpu as pltpu

ENCODING_DIM = 64
LANES = 128


def _body(pos_s, num_s, tableT_hbm, out_v, buf_v, sem):
    s = pos_s[0] % num_s[0]
    base = (s // LANES) * LANES
    r = s - base
    cp = pltpu.make_async_copy(tableT_hbm.at[:, pl.ds(base, LANES)], buf_v, sem)
    cp.start()
    lane = lax.broadcasted_iota(jnp.int32, (ENCODING_DIM, LANES), 1)
    mask = lane == r
    cp.wait()
    sel = jnp.where(mask, buf_v[...], 0.0)
    out_v[...] = jnp.sum(sel, axis=1)


def kernel(position, num_players, table):
    pos_arr = jnp.reshape(jnp.asarray(position, jnp.int32), (1,))
    num_arr = jnp.reshape(jnp.asarray(num_players, jnp.int32), (1,))
    table_t = table.T
    out = pl.pallas_call(
        _body,
        in_specs=[
            pl.BlockSpec(memory_space=pltpu.SMEM),
            pl.BlockSpec(memory_space=pltpu.SMEM),
            pl.BlockSpec(memory_space=pl.ANY),
        ],
        out_specs=pl.BlockSpec(memory_space=pltpu.VMEM),
        out_shape=jax.ShapeDtypeStruct((ENCODING_DIM,), jnp.float32),
        scratch_shapes=[
            pltpu.VMEM((ENCODING_DIM, LANES), jnp.float32),
            pltpu.SemaphoreType.DMA,
        ],
    )(pos_arr, num_arr, table_t)
    return out


# speedup vs baseline: 1.0048x; 1.0022x over previous
import jax
import jax.numpy as jnp
from jax import lax
from jax.experimental import pallas as pl
from jax.experimental.pallas import tpu as pltpu

ENCODING_DIM = 64
LANES = 128


def _body(s_ref, tableT_hbm, out_v, buf_v, sem):
    s = s_ref[0]
    base = (s // LANES) * LANES
    r = s - base
    cp = pltpu.make_async_copy(tableT_hbm.at[:, pl.ds(base, LANES)], buf_v, sem)
    cp.start()
    lane = lax.broadcasted_iota(jnp.int32, (ENCODING_DIM, LANES), 1)
    mask = lane == r
    cp.wait()
    sel = jnp.where(mask, buf_v[...], 0.0)
    out_v[...] = jnp.sum(sel, axis=1)


def kernel(position, num_players, table):
    s = jnp.asarray(position, jnp.int32) % jnp.asarray(num_players, jnp.int32)
    s_arr = jnp.reshape(s, (1,))
    table_t = table.T
    out = pl.pallas_call(
        _body,
        in_specs=[
            pl.BlockSpec(memory_space=pltpu.SMEM),
            pl.BlockSpec(memory_space=pl.ANY),
        ],
        out_specs=pl.BlockSpec(memory_space=pltpu.VMEM),
        out_shape=jax.ShapeDtypeStruct((ENCODING_DIM,), jnp.float32),
        scratch_shapes=[
            pltpu.VMEM((ENCODING_DIM, LANES), jnp.float32),
            pltpu.SemaphoreType.DMA,
        ],
    )(s_arr, table_t)
    return out
